# baseline (speedup 1.0000x reference)
import jax
import jax.numpy as jnp
from jax import lax
from jax.experimental import pallas as pl
from jax.experimental.pallas import tpu as pltpu
from jax.experimental.pallas import tpu_sc as plsc

V, E, D, DP = 10000, 320000, 128, 16
VP = 10240

NC, NS, L = 2, 16, 16
NW = NC * NS
C = 128
EP = 327680
NCH = EP // (NW * C)
G = C // L
NSLOT = 4
NBUF = 2
ZROWS = 16

BE = 3200


def _attn_body(fs_ref, npf_ref, w1_ref, w2_ref, b2_ref, w3_ref, b3_ref, o_ref):
    h = jnp.dot(fs_ref[...], w1_ref[...], preferred_element_type=jnp.float32)
    h = h + jnp.dot(npf_ref[...], w2_ref[...], preferred_element_type=jnp.float32)
    h = h + b2_ref[...]
    h = jnp.maximum(h, 0.0)
    s = jnp.dot(h, w3_ref[...], preferred_element_type=jnp.float32)
    o_ref[...] = jax.nn.sigmoid(s + b3_ref[...])


def _attention(feat_sum, node_pair_feat, W1, W2, b2, W3, b3):
    return pl.pallas_call(
        _attn_body,
        grid=(E // BE,),
        in_specs=[
            pl.BlockSpec((BE, D), lambda i: (i, 0)),
            pl.BlockSpec((BE, DP), lambda i: (i, 0)),
            pl.BlockSpec((D, D), lambda i: (0, 0)),
            pl.BlockSpec((DP, D), lambda i: (0, 0)),
            pl.BlockSpec((1, D), lambda i: (0, 0)),
            pl.BlockSpec((D, 1), lambda i: (0, 0)),
            pl.BlockSpec((1, 1), lambda i: (0, 0)),
        ],
        out_specs=pl.BlockSpec((BE, 1), lambda i: (i, 0)),
        out_shape=jax.ShapeDtypeStruct((E, 1), jnp.float32),
    )(feat_sum, node_pair_feat, W1, W2, b2.reshape(1, D), W3, b3.reshape(1, 1))


def _sc_body(nf_hbm, a_hbm, ei_hbm, out_hbm,
             sd_v, av_v, rows_v, zb_v, ctx_sh,
             sem_i0, sem_i1, sem_i2, sem_i3, sem_g0, sem_g1):
    c = lax.axis_index("c")
    s = lax.axis_index("s")
    wid = c * NS + s
    sem_i = [sem_i0, sem_i1, sem_i2, sem_i3]
    sem_g = [sem_g0, sem_g1]

    def idx_handles(j, slot):
        base = (wid * NCH + j) * C
        return (
            pltpu.make_async_copy(ei_hbm.at[:, pl.ds(base, C)],
                                  sd_v.at[slot], sem_i[slot]),
            pltpu.make_async_copy(a_hbm.at[pl.ds(base, C)],
                                  av_v.at[slot], sem_i[slot]),
        )

    def issue_idx(j, slot):
        for h in idx_handles(j, slot):
            h.start()

    def wait_idx(j, slot):
        for h in idx_handles(j, slot):
            h.wait()

    def gather_handle(slot, buf):
        return pltpu.make_async_copy(nf_hbm.at[sd_v.at[slot, 0]],
                                     rows_v.at[buf], sem_g[buf])

    def scale(slot, buf):
        def gbody(g, carry):
            a16 = av_v[slot, pl.ds(g * L, L)]
            for e in range(L):
                a_sp = lax.gather(
                    a16, jnp.full((L, 1), e, dtype=jnp.int32),
                    lax.GatherDimensionNumbers(offset_dims=(),
                                               collapsed_slice_dims=(0,),
                                               start_index_map=(0,)),
                    slice_sizes=(1,),
                    mode=lax.GatherScatterMode.PROMISE_IN_BOUNDS)
                row = g * L + e
                for jj in range(D // L):
                    sl = pl.ds(jj * L, L)
                    rows_v[buf, row, sl] = rows_v[buf, row, sl] * a_sp
            return carry

        lax.fori_loop(0, G, gbody, 0, unroll=False)

    for r in range(ZROWS):
        for jj in range(D // L):
            zb_v[r, pl.ds(jj * L, L)] = jnp.zeros((L,), jnp.float32)
    rows_per_sub = VP // NS
    stripe0 = s * rows_per_sub

    def zbody(z, carry):
        pltpu.sync_copy(zb_v, ctx_sh.at[pl.ds(stripe0 + z * ZROWS, ZROWS)])
        return carry

    lax.fori_loop(0, rows_per_sub // ZROWS, zbody, 0, unroll=False)
    plsc.subcore_barrier()

    def step(j, slot, buf, do_idx, do_gather):
        gather_handle(slot, buf).wait()
        if do_idx:
            issue_idx(j + 2, (slot + 2) % NSLOT)
        if do_gather:
            nslot = (slot + 1) % NSLOT
            wait_idx(j + 1, nslot)
            gather_handle(nslot, 1 - buf).start()
        scale(slot, buf)
        pltpu.sync_copy(rows_v.at[buf], ctx_sh.at[sd_v.at[slot, 1]], add=True)

    issue_idx(0, 0)
    issue_idx(1, 1)
    wait_idx(0, 0)
    gather_handle(0, 0).start()

    def kbody(k, carry):
        j0 = k * NSLOT
        for o in range(NSLOT):
            step(j0 + o, o, o % NBUF, True, True)
        return carry

    nk = (NCH - NSLOT - 1) // NSLOT
    lax.fori_loop(0, nk, kbody, 0, unroll=False)
    for j in range(nk * NSLOT, NCH):
        step(j, j % NSLOT, j % NBUF, j + 2 < NCH, j + 1 < NCH)

    plsc.subcore_barrier()
    pltpu.sync_copy(ctx_sh.at[pl.ds(stripe0, rows_per_sub)],
                    out_hbm.at[c, pl.ds(stripe0, rows_per_sub)])


def _sc_scatter(node_feats, a, edge_index):
    mesh = plsc.VectorSubcoreMesh(core_axis_name="c", subcore_axis_name="s")
    run = pl.kernel(
        _sc_body,
        out_type=jax.ShapeDtypeStruct((NC, VP, D), jnp.float32),
        mesh=mesh,
        scratch_types=[
            pltpu.VMEM((NSLOT, 2, C), jnp.int32),
            pltpu.VMEM((NSLOT, C), jnp.float32),
            pltpu.VMEM((NBUF, C, D), jnp.float32),
            pltpu.VMEM((ZROWS, D), jnp.float32),
            pltpu.VMEM_SHARED((VP, D), jnp.float32),
            pltpu.SemaphoreType.DMA,
            pltpu.SemaphoreType.DMA,
            pltpu.SemaphoreType.DMA,
            pltpu.SemaphoreType.DMA,
            pltpu.SemaphoreType.DMA,
            pltpu.SemaphoreType.DMA,
        ],
    )
    return run(node_feats, a, edge_index)


VB = 2000


def _add_body(p_ref, o_ref):
    o_ref[...] = p_ref[0] + p_ref[1]


def _add_partials(parts):
    return pl.pallas_call(
        _add_body,
        grid=(V // VB,),
        in_specs=[pl.BlockSpec((NC, VB, D), lambda i: (0, i, 0))],
        out_specs=pl.BlockSpec((VB, D), lambda i: (i, 0)),
        out_shape=jax.ShapeDtypeStruct((V, D), jnp.float32),
    )(parts)


def kernel(node_feats, feat_sum, node_pair_feat, W1, W2, b2, W3, b3, edge_index):
    a = _attention(feat_sum, node_pair_feat, W1, W2, b2, W3, b3)
    a_pad = jnp.pad(a.reshape(E), (0, EP - E))
    ei_pad = jnp.pad(edge_index, ((0, 0), (0, EP - E)))
    parts = _sc_scatter(node_feats, a_pad, ei_pad)
    return _add_partials(parts)


# speedup vs baseline: 1.2507x; 1.2507x over previous
import jax
import jax.numpy as jnp
from jax import lax
from jax.experimental import pallas as pl
from jax.experimental.pallas import tpu as pltpu
from jax.experimental.pallas import tpu_sc as plsc

V, E, D, DP = 10000, 320000, 128, 16
VP = 10240

NC, NS, L = 2, 16, 16
NW = NC * NS
C = 128
EP = 327680
NCH = EP // (NW * C)
G = C // L
NSLOT = 4
NBUF = 2
ZROWS = 16

BE = 3200


def _attn_body(fs_ref, npt_ref, w1_ref, w2_ref, b2_ref, w3_ref, b3_ref, o_ref):
    h = jnp.dot(fs_ref[...], w1_ref[...], preferred_element_type=jnp.float32)
    h = h + lax.dot_general(npt_ref[...], w2_ref[...], (((0,), (0,)), ((), ())),
                            preferred_element_type=jnp.float32)
    h = h + b2_ref[...]
    h = jnp.maximum(h, 0.0)
    s = jnp.dot(h, w3_ref[...], preferred_element_type=jnp.float32)
    o_ref[...] = jnp.reshape(jax.nn.sigmoid(s + b3_ref[...]), (1, BE // C, C))


def _attention(feat_sum, node_pair_feat_t, W1, W2, b2, W3, b3):
    return pl.pallas_call(
        _attn_body,
        grid=(E // BE,),
        in_specs=[
            pl.BlockSpec((BE, D), lambda i: (i, 0)),
            pl.BlockSpec((DP, BE), lambda i: (0, i)),
            pl.BlockSpec((D, D), lambda i: (0, 0)),
            pl.BlockSpec((DP, D), lambda i: (0, 0)),
            pl.BlockSpec((1, D), lambda i: (0, 0)),
            pl.BlockSpec((D, 1), lambda i: (0, 0)),
            pl.BlockSpec((1, 1), lambda i: (0, 0)),
        ],
        out_specs=pl.BlockSpec((1, BE // C, C), lambda i: (i, 0, 0)),
        out_shape=jax.ShapeDtypeStruct((E // BE, BE // C, C), jnp.float32),
    )(feat_sum, node_pair_feat_t, W1, W2, b2.reshape(1, D), W3, b3.reshape(1, 1))


def _sc_body(nf_hbm, a_hbm, ei_hbm, out_hbm,
             sd_v, av_v, rows_v, zb_v, ctx_sh,
             sem_i0, sem_i1, sem_i2, sem_i3, sem_g0, sem_g1, sem_s0, sem_s1):
    c = lax.axis_index("c")
    s = lax.axis_index("s")
    wid = c * NS + s
    sem_i = [sem_i0, sem_i1, sem_i2, sem_i3]
    sem_g = [sem_g0, sem_g1]
    sem_s = [sem_s0, sem_s1]

    def idx_handles(j, slot):
        base = (wid * NCH + j) * C
        return (
            pltpu.make_async_copy(ei_hbm.at[:, pl.ds(base, C)],
                                  sd_v.at[slot], sem_i[slot]),
            pltpu.make_async_copy(a_hbm.at[wid, j], av_v.at[slot], sem_i[slot]),
        )

    def issue_idx(j, slot):
        for h in idx_handles(j, slot):
            h.start()

    def wait_idx(j, slot):
        for h in idx_handles(j, slot):
            h.wait()

    def gather_handle(slot, buf):
        return pltpu.make_async_copy(nf_hbm.at[sd_v.at[slot, 0]],
                                     rows_v.at[buf], sem_g[buf])

    def scatter_wait(slot, buf):
        pltpu.make_async_copy(rows_v.at[buf], ctx_sh.at[sd_v.at[slot, 1]],
                              sem_s[buf]).wait()

    def scale(slot, buf):
        def gbody(g, carry):
            a16 = av_v[slot, pl.ds(g * L, L)]
            for e in range(L):
                a_sp = lax.gather(
                    a16, jnp.full((L, 1), e, dtype=jnp.int32),
                    lax.GatherDimensionNumbers(offset_dims=(),
                                               collapsed_slice_dims=(0,),
                                               start_index_map=(0,)),
                    slice_sizes=(1,),
                    mode=lax.GatherScatterMode.PROMISE_IN_BOUNDS)
                row = g * L + e
                for jj in range(D // L):
                    sl = pl.ds(jj * L, L)
                    rows_v[buf, row, sl] = rows_v[buf, row, sl] * a_sp
            return carry

        lax.fori_loop(0, G, gbody, 0, unroll=False)

    for r in range(ZROWS):
        for jj in range(D // L):
            zb_v[r, pl.ds(jj * L, L)] = jnp.zeros((L,), jnp.float32)
    rows_per_sub = VP // NS
    stripe0 = s * rows_per_sub

    def zbody(z, carry):
        pltpu.sync_copy(zb_v, ctx_sh.at[pl.ds(stripe0 + z * ZROWS, ZROWS)])
        return carry

    lax.fori_loop(0, rows_per_sub // ZROWS, zbody, 0, unroll=False)
    plsc.subcore_barrier()

    issue_idx(0, 0)
    issue_idx(1, 1)
    wait_idx(0, 0)
    gather_handle(0, 0).start()

    def kbody(k, carry):
        for o in range(NSLOT):
            j = k * NSLOT + o
            slot, buf = o, o % NBUF
            pslot, nslot = (o - 1) % NSLOT, (o + 1) % NSLOT
            gather_handle(slot, buf).wait()

            @pl.when(j + 2 < NCH)
            def _():
                issue_idx(j + 2, (o + 2) % NSLOT)

            scale(slot, buf)

            @pl.when((j >= 1) & (j + 1 < NCH))
            def _():
                scatter_wait(pslot, 1 - buf)

            @pl.when(j + 1 < NCH)
            def _():
                wait_idx(j + 1, nslot)
                gather_handle(nslot, 1 - buf).start()

            pltpu.async_copy(rows_v.at[buf], ctx_sh.at[sd_v.at[slot, 1]],
                             sem_s[buf], add=True)
        return carry

    lax.fori_loop(0, NCH // NSLOT, kbody, 0, unroll=False)
    scatter_wait((NCH - 2) % NSLOT, (NCH - 2) % NBUF)
    scatter_wait((NCH - 1) % NSLOT, (NCH - 1) % NBUF)

    plsc.subcore_barrier()
    pltpu.sync_copy(ctx_sh.at[pl.ds(stripe0, rows_per_sub)],
                    out_hbm.at[c, pl.ds(stripe0, rows_per_sub)])


def _sc_scatter(node_feats, a, edge_index):
    mesh = plsc.VectorSubcoreMesh(core_axis_name="c", subcore_axis_name="s")
    run = pl.kernel(
        _sc_body,
        out_type=jax.ShapeDtypeStruct((NC, VP, D), jnp.float32),
        mesh=mesh,
        scratch_types=[
            pltpu.VMEM((NSLOT, 2, C), jnp.int32),
            pltpu.VMEM((NSLOT, C), jnp.float32),
            pltpu.VMEM((NBUF, C, D), jnp.float32),
            pltpu.VMEM((ZROWS, D), jnp.float32),
            pltpu.VMEM_SHARED((VP, D), jnp.float32),
            pltpu.SemaphoreType.DMA,
            pltpu.SemaphoreType.DMA,
            pltpu.SemaphoreType.DMA,
            pltpu.SemaphoreType.DMA,
            pltpu.SemaphoreType.DMA,
            pltpu.SemaphoreType.DMA,
            pltpu.SemaphoreType.DMA,
            pltpu.SemaphoreType.DMA,
        ],
    )
    return run(node_feats, a, edge_index)


VB = 2000


def _add_body(p_ref, o_ref):
    o_ref[...] = p_ref[0] + p_ref[1]


def _add_partials(parts):
    return pl.pallas_call(
        _add_body,
        grid=(V // VB,),
        in_specs=[pl.BlockSpec((NC, VB, D), lambda i: (0, i, 0))],
        out_specs=pl.BlockSpec((VB, D), lambda i: (i, 0)),
        out_shape=jax.ShapeDtypeStruct((V, D), jnp.float32),
    )(parts)


def kernel(node_feats, feat_sum, node_pair_feat, W1, W2, b2, W3, b3, edge_index):
    a = _attention(feat_sum, node_pair_feat.T, W1, W2, b2, W3, b3)
    a3 = jnp.pad(a.reshape(E // C, C),
                 ((0, (EP - E) // C), (0, 0))).reshape(NW, NCH, C)
    ei_pad = jnp.pad(edge_index, ((0, 0), (0, EP - E)))
    parts = _sc_scatter(node_feats, a3, ei_pad)
    return _add_partials(parts)


# speedup vs baseline: 2.3821x; 1.9045x over previous
import jax
import jax.numpy as jnp
from jax import lax
from jax.experimental import pallas as pl
from jax.experimental.pallas import tpu as pltpu
from jax.experimental.pallas import tpu_sc as plsc

V, E, D, DP = 10000, 320000, 128, 16
VP = 10240

NC, NS, L = 2, 16, 16
NW = NC * NS
C = 128
EP = 327680
NCH = EP // (NW * C)
G = C // L
NSLOT = 4
NBUF = 2
ZROWS = 16

BE = 3200


def _attn_body(fs_ref, npt_ref, w1_ref, w2_ref, b2_ref, w3_ref, b3_ref, o_ref):
    h = jnp.dot(fs_ref[...], w1_ref[...], preferred_element_type=jnp.float32)
    h = h + lax.dot_general(npt_ref[...], w2_ref[...], (((0,), (0,)), ((), ())),
                            preferred_element_type=jnp.float32)
    h = h + b2_ref[...]
    h = jnp.maximum(h, 0.0)
    s = jnp.dot(h, w3_ref[...], preferred_element_type=jnp.float32)
    o_ref[...] = jnp.reshape(jax.nn.sigmoid(s + b3_ref[...]), (1, BE // C, C))


def _attention(feat_sum, node_pair_feat_t, W1, W2, b2, W3, b3):
    return pl.pallas_call(
        _attn_body,
        grid=(E // BE,),
        in_specs=[
            pl.BlockSpec((BE, D), lambda i: (i, 0)),
            pl.BlockSpec((DP, BE), lambda i: (0, i)),
            pl.BlockSpec((D, D), lambda i: (0, 0)),
            pl.BlockSpec((DP, D), lambda i: (0, 0)),
            pl.BlockSpec((1, D), lambda i: (0, 0)),
            pl.BlockSpec((D, 1), lambda i: (0, 0)),
            pl.BlockSpec((1, 1), lambda i: (0, 0)),
        ],
        out_specs=pl.BlockSpec((1, BE // C, C), lambda i: (i, 0, 0)),
        out_shape=jax.ShapeDtypeStruct((E // BE, BE // C, C), jnp.float32),
    )(feat_sum, node_pair_feat_t, W1, W2, b2.reshape(1, D), W3, b3.reshape(1, 1))


def _sc_body(nf_hbm, a_hbm, ei_hbm, out_hbm,
             sd_v, av_v, rows_v, zb_v, ctx_sh,
             sem_i0, sem_i1, sem_i2, sem_i3, sem_g0, sem_g1, sem_s0, sem_s1):
    c = lax.axis_index("c")
    s = lax.axis_index("s")
    wid = c * NS + s
    sem_i = [sem_i0, sem_i1, sem_i2, sem_i3]
    sem_g = [sem_g0, sem_g1]
    sem_s = [sem_s0, sem_s1]

    def idx_handles(j, slot):
        base = (wid * NCH + j) * C
        return (
            pltpu.make_async_copy(ei_hbm.at[:, pl.ds(base, C)],
                                  sd_v.at[slot], sem_i[slot]),
            pltpu.make_async_copy(a_hbm.at[wid, j], av_v.at[slot], sem_i[slot]),
        )

    def issue_idx(j, slot):
        for h in idx_handles(j, slot):
            h.start()

    def wait_idx(j, slot):
        for h in idx_handles(j, slot):
            h.wait()

    def gather_handle(slot, buf):
        return pltpu.make_async_copy(nf_hbm.at[sd_v.at[slot, 0]],
                                     rows_v.at[buf], sem_g[buf])

    def scatter_wait(slot, buf):
        pltpu.make_async_copy(rows_v.at[buf], ctx_sh.at[sd_v.at[slot, 1]],
                              sem_s[buf]).wait()

    def scale(slot, buf):
        def gbody(g, carry):
            a16 = av_v[slot, pl.ds(g * L, L)]
            for e in range(L):
                a_sp = lax.gather(
                    a16, jnp.full((L, 1), e, dtype=jnp.int32),
                    lax.GatherDimensionNumbers(offset_dims=(),
                                               collapsed_slice_dims=(0,),
                                               start_index_map=(0,)),
                    slice_sizes=(1,),
                    mode=lax.GatherScatterMode.PROMISE_IN_BOUNDS)
                row = g * L + e
                for jj in range(D // L):
                    sl = pl.ds(jj * L, L)
                    rows_v[buf, row, sl] = rows_v[buf, row, sl] * a_sp
            return carry

        lax.fori_loop(0, G, gbody, 0, unroll=False)

    for r in range(ZROWS):
        for jj in range(D // L):
            zb_v[r, pl.ds(jj * L, L)] = jnp.zeros((L,), jnp.float32)
    rows_per_sub = VP // NS
    stripe0 = s * rows_per_sub

    def zbody(z, carry):
        pltpu.sync_copy(zb_v, ctx_sh.at[pl.ds(stripe0 + z * ZROWS, ZROWS)])
        return carry

    lax.fori_loop(0, rows_per_sub // ZROWS, zbody, 0, unroll=False)
    plsc.subcore_barrier()

    issue_idx(0, 0)
    issue_idx(1, 1)
    wait_idx(0, 0)
    gather_handle(0, 0).start()

    def kbody(k, carry):
        for o in range(NSLOT):
            j = k * NSLOT + o
            slot, buf = o, o % NBUF
            pslot, nslot = (o - 1) % NSLOT, (o + 1) % NSLOT
            gather_handle(slot, buf).wait()

            @pl.when(j + 2 < NCH)
            def _():
                issue_idx(j + 2, (o + 2) % NSLOT)

            scale(slot, buf)

            @pl.when((j >= 1) & (j + 1 < NCH))
            def _():
                scatter_wait(pslot, 1 - buf)

            @pl.when(j + 1 < NCH)
            def _():
                wait_idx(j + 1, nslot)
                gather_handle(nslot, 1 - buf).start()

            pltpu.async_copy(rows_v.at[buf], ctx_sh.at[sd_v.at[slot, 1]],
                             sem_s[buf], add=True)
        return carry

    lax.fori_loop(0, NCH // NSLOT, kbody, 0, unroll=False)
    scatter_wait((NCH - 2) % NSLOT, (NCH - 2) % NBUF)
    scatter_wait((NCH - 1) % NSLOT, (NCH - 1) % NBUF)

    plsc.subcore_barrier()
    pltpu.sync_copy(ctx_sh.at[pl.ds(stripe0, rows_per_sub)],
                    out_hbm.at[c, pl.ds(stripe0, rows_per_sub)])


def _sc_scatter(node_feats, a, edge_index):
    mesh = plsc.VectorSubcoreMesh(core_axis_name="c", subcore_axis_name="s")
    run = pl.kernel(
        _sc_body,
        out_type=jax.ShapeDtypeStruct((NC, VP, D), jnp.float32),
        mesh=mesh,
        scratch_types=[
            pltpu.VMEM((NSLOT, 2, C), jnp.int32),
            pltpu.VMEM((NSLOT, C), jnp.float32),
            pltpu.VMEM((NBUF, C, D), jnp.float32),
            pltpu.VMEM((ZROWS, D), jnp.float32),
            pltpu.VMEM_SHARED((VP, D), jnp.float32),
            pltpu.SemaphoreType.DMA,
            pltpu.SemaphoreType.DMA,
            pltpu.SemaphoreType.DMA,
            pltpu.SemaphoreType.DMA,
            pltpu.SemaphoreType.DMA,
            pltpu.SemaphoreType.DMA,
            pltpu.SemaphoreType.DMA,
            pltpu.SemaphoreType.DMA,
        ],
    )
    return run(node_feats, a, edge_index)


VB = 2000


def _add_body(p_ref, o_ref):
    o_ref[...] = p_ref[0] + p_ref[1]


def _add_partials(parts):
    return pl.pallas_call(
        _add_body,
        grid=(V // VB,),
        in_specs=[pl.BlockSpec((NC, VB, D), lambda i: (0, i, 0))],
        out_specs=pl.BlockSpec((VB, D), lambda i: (i, 0)),
        out_shape=jax.ShapeDtypeStruct((V, D), jnp.float32),
    )(parts)


def kernel(node_feats, feat_sum, node_pair_feat, W1, W2, b2, W3, b3, edge_index):
    a = _attention(feat_sum, node_pair_feat.T, W1, W2, b2, W3, b3)
    a3 = jnp.pad(a.reshape(E // C, C),
                 ((0, (EP - E) // C), (0, 0))).reshape(NW, NCH, C)
    spread = jnp.arange(EP - E, dtype=jnp.int32) % (VP - V)
    ei_pad = jnp.concatenate(
        [edge_index, jnp.stack([spread, V + spread])], axis=1)
    parts = _sc_scatter(node_feats, a3, ei_pad)
    return _add_partials(parts)


# speedup vs baseline: 2.7589x; 1.1582x over previous
import jax
import jax.numpy as jnp
from jax import lax
from jax.experimental import pallas as pl
from jax.experimental.pallas import tpu as pltpu
from jax.experimental.pallas import tpu_sc as plsc

V, E, D, DP = 10000, 320000, 128, 16
VP = 10240

NC, NS, L = 2, 16, 16
NW = NC * NS
C = 128
NPH = 2
EH = E // NPH
EPH = 163840
NCH = EPH // (NW * C)
G = C // L
NSLOT = 4
NBUF = 2
ZROWS = 16

BE = 3200


def _attn_body(fs_ref, npt_ref, w1_ref, w2_ref, b2_ref, w3_ref, b3_ref, o_ref):
    h = jnp.dot(fs_ref[...], w1_ref[...], preferred_element_type=jnp.float32)
    h = h + lax.dot_general(npt_ref[...], w2_ref[...], (((0,), (0,)), ((), ())),
                            preferred_element_type=jnp.float32)
    h = h + b2_ref[...]
    h = jnp.maximum(h, 0.0)
    s = jnp.dot(h, w3_ref[...], preferred_element_type=jnp.float32)
    o_ref[...] = jnp.reshape(jax.nn.sigmoid(s + b3_ref[...]), (1, BE // C, C))


def _attention(p, feat_sum, node_pair_feat_t, W1, W2, b2, W3, b3):
    nb = EH // BE
    return pl.pallas_call(
        _attn_body,
        grid=(nb,),
        in_specs=[
            pl.BlockSpec((BE, D), lambda i: (i + p * nb, 0)),
            pl.BlockSpec((DP, BE), lambda i: (0, i + p * nb)),
            pl.BlockSpec((D, D), lambda i: (0, 0)),
            pl.BlockSpec((DP, D), lambda i: (0, 0)),
            pl.BlockSpec((1, D), lambda i: (0, 0)),
            pl.BlockSpec((D, 1), lambda i: (0, 0)),
            pl.BlockSpec((1, 1), lambda i: (0, 0)),
        ],
        out_specs=pl.BlockSpec((1, BE // C, C), lambda i: (i, 0, 0)),
        out_shape=jax.ShapeDtypeStruct((nb, BE // C, C), jnp.float32),
    )(feat_sum, node_pair_feat_t, W1, W2, b2.reshape(1, D), W3, b3.reshape(1, 1))


def _sc_body(nf_hbm, a_hbm, ei_hbm, out_hbm,
             sd_v, av_v, rows_v, zb_v, ctx_sh,
             sem_i0, sem_i1, sem_i2, sem_i3, sem_g0, sem_g1, sem_s0, sem_s1):
    c = lax.axis_index("c")
    s = lax.axis_index("s")
    wid = c * NS + s
    sem_i = [sem_i0, sem_i1, sem_i2, sem_i3]
    sem_g = [sem_g0, sem_g1]
    sem_s = [sem_s0, sem_s1]

    def idx_handles(j, slot):
        base = (wid * NCH + j) * C
        return (
            pltpu.make_async_copy(ei_hbm.at[:, pl.ds(base, C)],
                                  sd_v.at[slot], sem_i[slot]),
            pltpu.make_async_copy(a_hbm.at[wid, j], av_v.at[slot], sem_i[slot]),
        )

    def issue_idx(j, slot):
        for h in idx_handles(j, slot):
            h.start()

    def wait_idx(j, slot):
        for h in idx_handles(j, slot):
            h.wait()

    def gather_handle(slot, buf):
        return pltpu.make_async_copy(nf_hbm.at[sd_v.at[slot, 0]],
                                     rows_v.at[buf], sem_g[buf])

    def scatter_wait(slot, buf):
        pltpu.make_async_copy(rows_v.at[buf], ctx_sh.at[sd_v.at[slot, 1]],
                              sem_s[buf]).wait()

    def scale(slot, buf):
        def gbody(g, carry):
            a16 = av_v[slot, pl.ds(g * L, L)]
            for e in range(L):
                a_sp = lax.gather(
                    a16, jnp.full((L, 1), e, dtype=jnp.int32),
                    lax.GatherDimensionNumbers(offset_dims=(),
                                               collapsed_slice_dims=(0,),
                                               start_index_map=(0,)),
                    slice_sizes=(1,),
                    mode=lax.GatherScatterMode.PROMISE_IN_BOUNDS)
                row = g * L + e
                for jj in range(D // L):
                    sl = pl.ds(jj * L, L)
                    rows_v[buf, row, sl] = rows_v[buf, row, sl] * a_sp
            return carry

        lax.fori_loop(0, G, gbody, 0, unroll=False)

    for r in range(ZROWS):
        for jj in range(D // L):
            zb_v[r, pl.ds(jj * L, L)] = jnp.zeros((L,), jnp.float32)
    rows_per_sub = VP // NS
    stripe0 = s * rows_per_sub

    def zcopy(z):
        return pltpu.make_async_copy(
            zb_v, ctx_sh.at[pl.ds(stripe0 + z * ZROWS, ZROWS)], sem_s0)

    def zbody(z, carry):
        zcopy(z).start()
        return carry

    def zdrain(z, carry):
        zcopy(z).wait()
        return carry

    lax.fori_loop(0, rows_per_sub // ZROWS, zbody, 0, unroll=False)
    lax.fori_loop(0, rows_per_sub // ZROWS, zdrain, 0, unroll=False)
    plsc.subcore_barrier()

    issue_idx(0, 0)
    issue_idx(1, 1)
    wait_idx(0, 0)
    gather_handle(0, 0).start()

    def kbody(k, carry):
        for o in range(NSLOT):
            j = k * NSLOT + o
            slot, buf = o, o % NBUF
            pslot, nslot = (o - 1) % NSLOT, (o + 1) % NSLOT
            gather_handle(slot, buf).wait()

            @pl.when(j + 2 < NCH)
            def _():
                issue_idx(j + 2, (o + 2) % NSLOT)

            scale(slot, buf)

            @pl.when((j >= 1) & (j + 1 < NCH))
            def _():
                scatter_wait(pslot, 1 - buf)

            @pl.when(j + 1 < NCH)
            def _():
                wait_idx(j + 1, nslot)
                gather_handle(nslot, 1 - buf).start()

            pltpu.async_copy(rows_v.at[buf], ctx_sh.at[sd_v.at[slot, 1]],
                             sem_s[buf], add=True)
        return carry

    lax.fori_loop(0, NCH // NSLOT, kbody, 0, unroll=False)
    scatter_wait((NCH - 2) % NSLOT, (NCH - 2) % NBUF)
    scatter_wait((NCH - 1) % NSLOT, (NCH - 1) % NBUF)

    plsc.subcore_barrier()
    pltpu.sync_copy(ctx_sh.at[pl.ds(stripe0, rows_per_sub)],
                    out_hbm.at[c, pl.ds(stripe0, rows_per_sub)])


def _sc_scatter(node_feats, a, edge_index):
    mesh = plsc.VectorSubcoreMesh(core_axis_name="c", subcore_axis_name="s")
    run = pl.kernel(
        _sc_body,
        out_type=jax.ShapeDtypeStruct((NC, VP, D), jnp.float32),
        mesh=mesh,
        scratch_types=[
            pltpu.VMEM((NSLOT, 2, C), jnp.int32),
            pltpu.VMEM((NSLOT, C), jnp.float32),
            pltpu.VMEM((NBUF, C, D), jnp.float32),
            pltpu.VMEM((ZROWS, D), jnp.float32),
            pltpu.VMEM_SHARED((VP, D), jnp.float32),
            pltpu.SemaphoreType.DMA,
            pltpu.SemaphoreType.DMA,
            pltpu.SemaphoreType.DMA,
            pltpu.SemaphoreType.DMA,
            pltpu.SemaphoreType.DMA,
            pltpu.SemaphoreType.DMA,
            pltpu.SemaphoreType.DMA,
            pltpu.SemaphoreType.DMA,
        ],
    )
    return run(node_feats, a, edge_index)


VB = 2000


def _add_body(p_ref, q_ref, o_ref):
    o_ref[...] = (p_ref[0] + p_ref[1]) + (q_ref[0] + q_ref[1])


def _add_partials(parts0, parts1):
    return pl.pallas_call(
        _add_body,
        grid=(V // VB,),
        in_specs=[pl.BlockSpec((NC, VB, D), lambda i: (0, i, 0)),
                  pl.BlockSpec((NC, VB, D), lambda i: (0, i, 0))],
        out_specs=pl.BlockSpec((VB, D), lambda i: (i, 0)),
        out_shape=jax.ShapeDtypeStruct((V, D), jnp.float32),
    )(parts0, parts1)


def _phase_inputs(p, a, edge_index):
    a3 = jnp.pad(a.reshape(EH // C, C),
                 ((0, (EPH - EH) // C), (0, 0))).reshape(NW, NCH, C)
    spread = jnp.arange(EPH - EH, dtype=jnp.int32) % (VP - V)
    ei = lax.slice(edge_index, (0, p * EH), (2, (p + 1) * EH))
    ei_pad = jnp.concatenate([ei, jnp.stack([spread, V + spread])], axis=1)
    return a3, ei_pad


def kernel(node_feats, feat_sum, node_pair_feat, W1, W2, b2, W3, b3, edge_index):
    npt = node_pair_feat.T
    parts = []
    for p in range(NPH):
        a = _attention(p, feat_sum, npt, W1, W2, b2, W3, b3)
        a3, ei_pad = _phase_inputs(p, a, edge_index)
        parts.append(_sc_scatter(node_feats, a3, ei_pad))
    return _add_partials(*parts)


# speedup vs baseline: 3.2034x; 1.1611x over previous
import jax
import jax.numpy as jnp
from jax import lax
from jax.experimental import pallas as pl
from jax.experimental.pallas import tpu as pltpu
from jax.experimental.pallas import tpu_sc as plsc

V, E, D, DP = 10000, 320000, 128, 16
VP = 10240

NC, NS, L = 2, 16, 16
NW = NC * NS
C = 128
NPH = 2
EH = E // NPH
EPH = 163840
NCH = EPH // (NW * C)
G = C // L
NSLOT = 4
NBUF = 2
ZROWS = 16

BE = 3200


def _attn_body(fs_ref, npt_ref, w1_ref, w2_ref, b2_ref, w3_ref, b3_ref, o_ref):
    h = jnp.dot(fs_ref[...], w1_ref[...], preferred_element_type=jnp.float32)
    h = h + lax.dot_general(npt_ref[...], w2_ref[...], (((0,), (0,)), ((), ())),
                            preferred_element_type=jnp.float32)
    h = h + b2_ref[...]
    h = jnp.maximum(h, 0.0)
    s = jnp.dot(h, w3_ref[...], preferred_element_type=jnp.float32)
    o_ref[...] = jnp.reshape(jax.nn.sigmoid(s + b3_ref[...]), (1, BE // C, C))


def _attention(p, feat_sum, node_pair_feat_t, W1, W2, b2, W3, b3):
    nb = EH // BE
    return pl.pallas_call(
        _attn_body,
        grid=(nb,),
        in_specs=[
            pl.BlockSpec((BE, D), lambda i: (i + p * nb, 0)),
            pl.BlockSpec((DP, BE), lambda i: (0, i + p * nb)),
            pl.BlockSpec((D, D), lambda i: (0, 0)),
            pl.BlockSpec((DP, D), lambda i: (0, 0)),
            pl.BlockSpec((1, D), lambda i: (0, 0)),
            pl.BlockSpec((D, 1), lambda i: (0, 0)),
            pl.BlockSpec((1, 1), lambda i: (0, 0)),
        ],
        out_specs=pl.BlockSpec((1, BE // C, C), lambda i: (i, 0, 0)),
        out_shape=jax.ShapeDtypeStruct((nb, BE // C, C), jnp.float32),
    )(feat_sum, node_pair_feat_t, W1, W2, b2.reshape(1, D), W3, b3.reshape(1, 1))


def _sc_body(nf_hbm, a_hbm, ei_hbm, out_hbm,
             sd_v, av_v, rows_v, zb_v, ctx_sh,
             sem_i0, sem_i1, sem_i2, sem_i3, sem_g0, sem_g1, sem_s0, sem_s1):
    c = lax.axis_index("c")
    s = lax.axis_index("s")
    wid = c * NS + s
    sem_i = [sem_i0, sem_i1, sem_i2, sem_i3]
    sem_g = [sem_g0, sem_g1]
    sem_s = [sem_s0, sem_s1]

    def idx_handles(j, slot):
        base = (wid * NCH + j) * C
        return (
            pltpu.make_async_copy(ei_hbm.at[:, pl.ds(base, C)],
                                  sd_v.at[slot], sem_i[slot]),
            pltpu.make_async_copy(a_hbm.at[wid, j], av_v.at[slot], sem_i[slot]),
        )

    def issue_idx(j, slot):
        for h in idx_handles(j, slot):
            h.start()

    def wait_idx(j, slot):
        for h in idx_handles(j, slot):
            h.wait()

    def gather_handle(slot, buf):
        return pltpu.make_async_copy(nf_hbm.at[sd_v.at[slot, 0]],
                                     rows_v.at[buf], sem_g[buf])

    def scatter_wait(slot, buf):
        pltpu.make_async_copy(rows_v.at[buf], ctx_sh.at[sd_v.at[slot, 1]],
                              sem_s[buf]).wait()

    def scale(slot, buf):
        def gbody(g, carry):
            a16 = av_v[slot, pl.ds(g * L, L)]
            for e in range(L):
                a_sp = lax.gather(
                    a16, jnp.full((L, 1), e, dtype=jnp.int32),
                    lax.GatherDimensionNumbers(offset_dims=(),
                                               collapsed_slice_dims=(0,),
                                               start_index_map=(0,)),
                    slice_sizes=(1,),
                    mode=lax.GatherScatterMode.PROMISE_IN_BOUNDS)
                row = g * L + e
                for jj in range(D // L):
                    sl = pl.ds(jj * L, L)
                    rows_v[buf, row, sl] = rows_v[buf, row, sl] * a_sp
            return carry

        lax.fori_loop(0, G, gbody, 0, unroll=False)

    for r in range(ZROWS):
        for jj in range(D // L):
            zb_v[r, pl.ds(jj * L, L)] = jnp.zeros((L,), jnp.float32)
    rows_per_sub = VP // NS
    stripe0 = s * rows_per_sub

    def zcopy(z):
        return pltpu.make_async_copy(
            zb_v, ctx_sh.at[pl.ds(stripe0 + z * ZROWS, ZROWS)], sem_s0)

    def zbody(z, carry):
        zcopy(z).start()
        return carry

    def zdrain(z, carry):
        zcopy(z).wait()
        return carry

    lax.fori_loop(0, rows_per_sub // ZROWS, zbody, 0, unroll=False)
    lax.fori_loop(0, rows_per_sub // ZROWS, zdrain, 0, unroll=False)
    plsc.subcore_barrier()

    issue_idx(0, 0)
    issue_idx(1, 1)
    wait_idx(0, 0)
    gather_handle(0, 0).start()

    def kbody(k, carry):
        for o in range(NSLOT):
            j = k * NSLOT + o
            slot, buf = o, o % NBUF
            pslot, nslot = (o - 1) % NSLOT, (o + 1) % NSLOT
            gather_handle(slot, buf).wait()

            @pl.when((j >= 1) & (j + 1 < NCH))
            def _():
                scatter_wait(pslot, 1 - buf)

            @pl.when(j + 1 < NCH)
            def _():
                wait_idx(j + 1, nslot)
                gather_handle(nslot, 1 - buf).start()

            @pl.when(j + 2 < NCH)
            def _():
                issue_idx(j + 2, (o + 2) % NSLOT)

            scale(slot, buf)
            pltpu.async_copy(rows_v.at[buf], ctx_sh.at[sd_v.at[slot, 1]],
                             sem_s[buf], add=True)
        return carry

    lax.fori_loop(0, NCH // NSLOT, kbody, 0, unroll=False)
    scatter_wait((NCH - 2) % NSLOT, (NCH - 2) % NBUF)
    scatter_wait((NCH - 1) % NSLOT, (NCH - 1) % NBUF)

    plsc.subcore_barrier()
    pltpu.sync_copy(ctx_sh.at[pl.ds(stripe0, rows_per_sub)],
                    out_hbm.at[c, pl.ds(stripe0, rows_per_sub)])


def _sc_scatter(node_feats, a, edge_index):
    mesh = plsc.VectorSubcoreMesh(core_axis_name="c", subcore_axis_name="s")
    run = pl.kernel(
        _sc_body,
        out_type=jax.ShapeDtypeStruct((NC, VP, D), jnp.float32),
        mesh=mesh,
        scratch_types=[
            pltpu.VMEM((NSLOT, 2, C), jnp.int32),
            pltpu.VMEM((NSLOT, C), jnp.float32),
            pltpu.VMEM((NBUF, C, D), jnp.float32),
            pltpu.VMEM((ZROWS, D), jnp.float32),
            pltpu.VMEM_SHARED((VP, D), jnp.float32),
            pltpu.SemaphoreType.DMA,
            pltpu.SemaphoreType.DMA,
            pltpu.SemaphoreType.DMA,
            pltpu.SemaphoreType.DMA,
            pltpu.SemaphoreType.DMA,
            pltpu.SemaphoreType.DMA,
            pltpu.SemaphoreType.DMA,
            pltpu.SemaphoreType.DMA,
        ],
    )
    return run(node_feats, a, edge_index)


VB = 2000


def _add_body(p_ref, q_ref, o_ref):
    o_ref[...] = (p_ref[0] + p_ref[1]) + (q_ref[0] + q_ref[1])


def _add_partials(parts0, parts1):
    return pl.pallas_call(
        _add_body,
        grid=(V // VB,),
        in_specs=[pl.BlockSpec((NC, VB, D), lambda i: (0, i, 0)),
                  pl.BlockSpec((NC, VB, D), lambda i: (0, i, 0))],
        out_specs=pl.BlockSpec((VB, D), lambda i: (i, 0)),
        out_shape=jax.ShapeDtypeStruct((V, D), jnp.float32),
    )(parts0, parts1)


def _phase_inputs(p, a, edge_index):
    a3 = jnp.pad(a.reshape(EH // C, C),
                 ((0, (EPH - EH) // C), (0, 0))).reshape(NW, NCH, C)
    spread = jnp.arange(EPH - EH, dtype=jnp.int32) % (VP - V)
    ei = lax.slice(edge_index, (0, p * EH), (2, (p + 1) * EH))
    ei_pad = jnp.concatenate([ei, jnp.stack([spread, V + spread])], axis=1)
    return a3, ei_pad


def kernel(node_feats, feat_sum, node_pair_feat, W1, W2, b2, W3, b3, edge_index):
    npt = node_pair_feat.T
    parts = []
    for p in range(NPH):
        a = _attention(p, feat_sum, npt, W1, W2, b2, W3, b3)
        a3, ei_pad = _phase_inputs(p, a, edge_index)
        parts.append(_sc_scatter(node_feats, a3, ei_pad))
    return _add_partials(*parts)
